# baseline (speedup 1.0000x reference)
import functools

import jax
import jax.numpy as jnp
from jax import lax
from jax.experimental import pallas as pl
from jax.experimental.pallas import tpu as pltpu
from jax.experimental.pallas import tpu_sc as plsc

_CHUNK_ROWS = 32
_NBUF = 2


def kernel(x, table):
    seq_length = x.shape[1]
    num_rows, hidden = table.shape
    seq_length = min(seq_length, num_rows)

    info = plsc.get_sparse_core_info()
    num_workers = info.num_cores * info.num_subcores
    rows_per_w = seq_length // num_workers
    assert rows_per_w * num_workers == seq_length
    assert rows_per_w % (2 * _CHUNK_ROWS) == 0
    n_per_path = rows_per_w // (2 * _CHUNK_ROWS)

    mesh = plsc.VectorSubcoreMesh(core_axis_name="c", subcore_axis_name="s")

    @functools.partial(
        pl.kernel,
        mesh=mesh,
        out_type=jax.ShapeDtypeStruct((seq_length, hidden), table.dtype),
        scratch_types=[
            pltpu.VMEM((_NBUF, _CHUNK_ROWS, hidden), table.dtype),
            pltpu.VMEM_SHARED(
                (info.num_subcores, _NBUF, _CHUNK_ROWS, hidden), table.dtype
            ),
            pltpu.SemaphoreType.DMA((_NBUF,)),
            pltpu.SemaphoreType.DMA((_NBUF,)),
            pltpu.SemaphoreType.DMA((_NBUF,)),
            pltpu.SemaphoreType.DMA((_NBUF,)),
        ],
    )
    def copy_rows(table_hbm, out_hbm, tbuf, shared, tl, ts, sl, ss):
        sid = lax.axis_index("s")
        wid = sid * info.num_cores + lax.axis_index("c")
        base = wid * rows_per_w

        def rows_at(g):
            return pl.ds(base + g * _CHUNK_ROWS, _CHUNK_ROWS)

        def load_a(g, b):
            return pltpu.async_copy(table_hbm.at[rows_at(2 * g)], tbuf.at[b], tl.at[b])

        def store_a(g, b):
            return pltpu.async_copy(tbuf.at[b], out_hbm.at[rows_at(2 * g)], ts.at[b])

        def load_b(g, b):
            return pltpu.async_copy(
                table_hbm.at[rows_at(2 * g + 1)], shared.at[sid, b], sl.at[b]
            )

        def store_b(g, b):
            return pltpu.async_copy(
                shared.at[sid, b], out_hbm.at[rows_at(2 * g + 1)], ss.at[b]
            )

        la = [load_a(g, g) for g in range(min(_NBUF, n_per_path))]
        lb = [load_b(g, g) for g in range(min(_NBUF, n_per_path))]
        sa = [None] * _NBUF
        sb = [None] * _NBUF
        for g in range(n_per_path):
            b = g % _NBUF
            nxt = g + _NBUF
            la[b].wait()
            sa[b] = store_a(g, b)
            lb[b].wait()
            sb[b] = store_b(g, b)
            if nxt < n_per_path:
                sa[b].wait()
                sa[b] = None
                la[b] = load_a(nxt, b)
                sb[b].wait()
                sb[b] = None
                lb[b] = load_b(nxt, b)
        for h in sa + sb:
            if h is not None:
                h.wait()

    return copy_rows(table)
